# Initial kernel scaffold; baseline (speedup 1.0000x reference)
import jax
import jax.numpy as jnp
from jax.experimental import pallas as pl


def kernel(x, emb_table, lin_table, lin_bias, W1, b1, W2, b2, W3, b3):
    raise NotImplementedError("write your pallas kernel here")


# speedup vs baseline: 3.3677x; 3.3677x over previous
import functools

import jax
import jax.numpy as jnp
from jax import lax
from jax.experimental import pallas as pl
from jax.experimental.pallas import tpu as pltpu
from jax.experimental.pallas import tpu_sc as plsc

NUM_FIELDS = 26
FIELD_DIM = 38462
EMBED_DIM = 16
BATCH = 16384
TOTAL = NUM_FIELDS * BATCH
VOCAB = NUM_FIELDS * FIELD_DIM
VPAD = 1000016
NC, NS = 2, 16
NW = NC * NS
ROWS_PER_STREAM = 128
NBLK = TOTAL // ROWS_PER_STREAM
BLK_PER_W = NBLK // NW
CHUNK_BLKS = 8
NCHUNK = BLK_PER_W // CHUNK_BLKS

TCOLS = 2048
NTCH = VOCAB // TCOLS
TAIL_START = NTCH * TCOLS
TAIL = 512
TAIL_WORKER = NTCH % NW
XTAIL_START = TAIL_START + TAIL
XTAIL = VOCAB - XTAIL_START


def _sc_transpose(embT):
    mesh = plsc.VectorSubcoreMesh(core_axis_name="c", subcore_axis_name="s")

    @functools.partial(
        pl.kernel,
        out_type=jax.ShapeDtypeStruct((VPAD * EMBED_DIM,), jnp.float32),
        mesh=mesh,
        scratch_types=[
            pltpu.VMEM((EMBED_DIM, TCOLS), jnp.float32),
            pltpu.VMEM((TCOLS * EMBED_DIM,), jnp.float32),
        ],
        compiler_params=pltpu.CompilerParams(needs_layout_passes=False),
    )
    def k(src_hbm, out_hbm, src_v, dst_v):
        wid = lax.axis_index("s") * NC + lax.axis_index("c")
        lane = lax.iota(jnp.int32, 16) * EMBED_DIM

        def do_chunk(col0, ncolgrp, ncols_dma):
            pltpu.sync_copy(src_hbm.at[:, pl.ds(col0, ncols_dma)],
                            src_v.at[:, pl.ds(0, ncols_dma)])

            def grp(j, carry):
                for d in range(EMBED_DIM):
                    v = src_v[d, pl.ds(j * 16, 16)]
                    plsc.store_scatter(dst_v, [lane + (j * 256 + d)], v)
                return carry

            lax.fori_loop(0, ncolgrp, grp, 0)
            pltpu.sync_copy(dst_v.at[pl.ds(0, ncolgrp * 256)],
                            out_hbm.at[pl.ds(col0 * EMBED_DIM,
                                             ncolgrp * 256)])

        def body(i, carry):
            cid = i * NW + wid

            @pl.when(cid < NTCH)
            def _():
                do_chunk(cid * TCOLS, TCOLS // 16, TCOLS)

            return carry

        lax.fori_loop(0, (NTCH + NW - 1) // NW, body, 0)

        @pl.when(wid == TAIL_WORKER)
        def _():
            do_chunk(TAIL_START, TAIL // 16, TAIL)

    return k(embT)


def _sc_gather(idx2, emb_lin, lin_flat):
    mesh = plsc.VectorSubcoreMesh(core_axis_name="c", subcore_axis_name="s")

    @functools.partial(
        pl.kernel,
        out_type=(
            jax.ShapeDtypeStruct((NBLK, ROWS_PER_STREAM, EMBED_DIM), jnp.float32),
            jax.ShapeDtypeStruct((NBLK, ROWS_PER_STREAM), jnp.float32),
        ),
        mesh=mesh,
        scratch_types=[
            pltpu.VMEM((CHUNK_BLKS, ROWS_PER_STREAM), jnp.int32),
            pltpu.VMEM((CHUNK_BLKS, ROWS_PER_STREAM, EMBED_DIM), jnp.float32),
            pltpu.VMEM((CHUNK_BLKS, ROWS_PER_STREAM), jnp.float32),
            pltpu.SemaphoreType.DMA,
            pltpu.SemaphoreType.DMA,
        ],
        compiler_params=pltpu.CompilerParams(use_tc_tiling_on_sc=False),
    )
    def k(idx_hbm, emb_hbm, lin_hbm, emb_out, lin_out, idx_v, rows_v, lin_v,
          sem_e, sem_l):
        wid = lax.axis_index("s") * NC + lax.axis_index("c")
        base = wid * BLK_PER_W

        def body(c, carry):
            off = base + c * CHUNK_BLKS
            pltpu.sync_copy(idx_hbm.at[pl.ds(off, CHUNK_BLKS)], idx_v)
            embs = [pltpu.async_copy(emb_hbm.at[idx_v.at[j]], rows_v.at[j], sem_e)
                    for j in range(CHUNK_BLKS)]
            lins = [pltpu.async_copy(lin_hbm.at[idx_v.at[j]], lin_v.at[j], sem_l)
                    for j in range(CHUNK_BLKS)]
            for cp in embs:
                cp.wait()
            for cp in lins:
                cp.wait()
            pltpu.sync_copy(rows_v, emb_out.at[pl.ds(off, CHUNK_BLKS)])
            pltpu.sync_copy(lin_v, lin_out.at[pl.ds(off, CHUNK_BLKS)])
            return carry

        lax.fori_loop(0, NCHUNK, body, 0)

    return k(idx2, emb_lin, lin_flat)


BB = 1024
MB = BB // 8
GRID = BATCH // BB


def _tc_body(embp_ref, linp_ref, bd1_ref, b1t_ref, bd2_ref, b2t_ref, bd3_ref,
             ones_ref, cb_ref, out_ref):
    e = embp_ref[...]
    s = jnp.sum(e, axis=0)
    s2 = jnp.sum(e * e, axis=0)
    g = s * s - s2
    fm = 0.5 * jnp.dot(g.astype(jnp.bfloat16), ones_ref[...],
                       preferred_element_type=jnp.float32)
    eb = e.reshape(NUM_FIELDS * MB, 128).astype(jnp.bfloat16)
    h = jnp.dot(eb, bd1_ref[...], preferred_element_type=jnp.float32)
    h = jnp.maximum(h + b1t_ref[...], 0.0)
    h2 = jnp.dot(h.astype(jnp.bfloat16), bd2_ref[...],
                 preferred_element_type=jnp.float32)
    h2 = jnp.maximum(h2 + b2t_ref[...], 0.0)
    h2s = jnp.sum(h2.reshape(NUM_FIELDS, MB, 256), axis=0)
    mlp = jnp.dot(h2s.astype(jnp.bfloat16), bd3_ref[...],
                  preferred_element_type=jnp.float32)
    lin = jnp.sum(linp_ref[...], axis=0)
    logits = lin + fm + mlp + cb_ref[0, 0]
    out_ref[...] = jax.nn.sigmoid(logits)


def _tc_dense(embp, linp, bd1, b1t, bd2, b2t, bd3, onesbd, cb):
    return pl.pallas_call(
        _tc_body,
        grid=(GRID,),
        in_specs=[
            pl.BlockSpec((NUM_FIELDS, MB, 128), lambda i: (0, i, 0)),
            pl.BlockSpec((NUM_FIELDS, MB, 8), lambda i: (0, i, 0)),
            pl.BlockSpec((128, 512), lambda i: (0, 0)),
            pl.BlockSpec((1, 512), lambda i: (0, 0)),
            pl.BlockSpec((512, 256), lambda i: (0, 0)),
            pl.BlockSpec((1, 256), lambda i: (0, 0)),
            pl.BlockSpec((256, 8), lambda i: (0, 0)),
            pl.BlockSpec((128, 8), lambda i: (0, 0)),
            pl.BlockSpec((1, 1), lambda i: (0, 0)),
        ],
        out_specs=pl.BlockSpec((MB, 8), lambda i: (i, 0)),
        out_shape=jax.ShapeDtypeStruct((BATCH // 8, 8), jnp.float32),
    )(embp, linp, bd1, b1t, bd2, b2t, bd3, onesbd, cb)


def kernel(x, emb_table, lin_table, lin_bias, W1, b1, W2, b2, W3, b3):
    offsets = FIELD_DIM * jnp.arange(NUM_FIELDS, dtype=jnp.int32)
    idx = (x.astype(jnp.int32).T + offsets[:, None]).reshape(NBLK, ROWS_PER_STREAM)
    lin_flat = lin_table.reshape(-1)

    emb_flat = _sc_transpose(emb_table.T)
    xtail = emb_table[XTAIL_START:VOCAB].reshape(XTAIL * EMBED_DIM)
    emb_flat = lax.dynamic_update_slice(emb_flat, xtail,
                                        (XTAIL_START * EMBED_DIM,))
    emb_lin = emb_flat.reshape(VPAD, EMBED_DIM)
    emb_rows, lin_rows = _sc_gather(idx, emb_lin, lin_flat)
    embp = emb_rows.reshape(NUM_FIELDS, BATCH // 8, 128)
    linp = lin_rows.reshape(NUM_FIELDS, BATCH // 8, 8)

    eye8 = jnp.eye(8, dtype=jnp.float32)
    bd1 = jnp.kron(eye8, W1).astype(jnp.bfloat16)
    bd2 = jnp.kron(eye8, W2).astype(jnp.bfloat16)
    bd3 = jnp.kron(eye8, W3).astype(jnp.bfloat16)
    onesbd = jnp.kron(eye8, jnp.ones((EMBED_DIM, 1), jnp.float32)).astype(
        jnp.bfloat16)
    b1t = jnp.tile(b1, 8).reshape(1, 512)
    b2t = jnp.tile(b2, 8).reshape(1, 256)
    cb = (lin_bias[0] + NUM_FIELDS * b3[0]).reshape(1, 1)

    out8 = _tc_dense(embp, linp, bd1, b1t, bd2, b2t, bd3, onesbd, cb)
    return out8.reshape(BATCH)


# speedup vs baseline: 3.9564x; 1.1748x over previous
import functools

import jax
import jax.numpy as jnp
from jax import lax
from jax.experimental import pallas as pl
from jax.experimental.pallas import tpu as pltpu
from jax.experimental.pallas import tpu_sc as plsc

NUM_FIELDS = 26
FIELD_DIM = 38462
EMBED_DIM = 16
BATCH = 16384
TOTAL = NUM_FIELDS * BATCH
VOCAB = NUM_FIELDS * FIELD_DIM
VPAD = 1000016
NC, NS = 2, 16
NW = NC * NS
ROWS_PER_STREAM = 128
NBLK = TOTAL // ROWS_PER_STREAM
BLK_PER_W = NBLK // NW
CHUNK_BLKS = 8
NCHUNK = BLK_PER_W // CHUNK_BLKS


TCOLS = 1536
NTCH = VOCAB // TCOLS
XTAIL_START = NTCH * TCOLS
XTAIL = VOCAB - XTAIL_START
NFULL_IT = NTCH // NW


def _sc_transpose(embT, xtail):
    mesh = plsc.VectorSubcoreMesh(core_axis_name="c", subcore_axis_name="s")

    @functools.partial(
        pl.kernel,
        out_type=jax.ShapeDtypeStruct((VPAD * EMBED_DIM,), jnp.float32),
        mesh=mesh,
        scratch_types=[
            pltpu.VMEM((EMBED_DIM, TCOLS), jnp.float32),
            pltpu.VMEM((EMBED_DIM, TCOLS), jnp.float32),
            pltpu.VMEM((TCOLS * EMBED_DIM,), jnp.float32),
            pltpu.VMEM((TCOLS * EMBED_DIM,), jnp.float32),
            pltpu.VMEM((XTAIL * EMBED_DIM,), jnp.float32),
            pltpu.SemaphoreType.DMA,
            pltpu.SemaphoreType.DMA,
            pltpu.SemaphoreType.DMA,
            pltpu.SemaphoreType.DMA,
        ],
        compiler_params=pltpu.CompilerParams(needs_layout_passes=False),
    )
    def k(src_hbm, xtail_hbm, out_hbm, src_v0, src_v1, dst_v0, dst_v1, xt_v,
          sem_in0, sem_in1, sem_out0, sem_out1):
        wid = lax.axis_index("s") * NC + lax.axis_index("c")
        lane = lax.iota(jnp.int32, 16) * EMBED_DIM
        src_v = (src_v0, src_v1)
        dst_v = (dst_v0, dst_v1)
        sem_in = (sem_in0, sem_in1)
        sem_out = (sem_out0, sem_out1)

        def transpose_buf(b, ncolgrp):
            def grp(j, carry):
                for d in range(EMBED_DIM):
                    v = src_v[b][d, pl.ds(j * 16, 16)]
                    plsc.store_scatter(dst_v[b], [lane + (j * 256 + d)], v)
                return carry

            lax.fori_loop(0, ncolgrp, grp, 0)

        def start_in(i):
            col0 = (i * NW + wid) * TCOLS
            return pltpu.async_copy(src_hbm.at[:, pl.ds(col0, TCOLS)],
                                    src_v[i % 2], sem_in[i % 2])

        cp_in = [None, None]
        cp_out = [None, None]
        cp_in[0] = start_in(0)
        for i in range(NFULL_IT):
            b = i % 2
            if i + 1 < NFULL_IT:
                cp_in[1 - b] = start_in(i + 1)
            cp_in[b].wait()
            if cp_out[b] is not None:
                cp_out[b].wait()
            transpose_buf(b, TCOLS // 16)
            col0 = (i * NW + wid) * TCOLS
            cp_out[b] = pltpu.async_copy(
                dst_v[b],
                out_hbm.at[pl.ds(col0 * EMBED_DIM, TCOLS * EMBED_DIM)],
                sem_out[b])
        for cp in cp_out:
            if cp is not None:
                cp.wait()

        @pl.when(wid < NTCH - NFULL_IT * NW)
        def _():
            col0 = (NFULL_IT * NW + wid) * TCOLS
            pltpu.sync_copy(src_hbm.at[:, pl.ds(col0, TCOLS)], src_v0)
            transpose_buf(0, TCOLS // 16)
            pltpu.sync_copy(dst_v0,
                            out_hbm.at[pl.ds(col0 * EMBED_DIM,
                                             TCOLS * EMBED_DIM)])

        @pl.when(wid == NTCH - NFULL_IT * NW)
        def _():
            pltpu.sync_copy(xtail_hbm, xt_v)
            pltpu.sync_copy(xt_v, out_hbm.at[pl.ds(XTAIL_START * EMBED_DIM,
                                                   XTAIL * EMBED_DIM)])

    return k(embT, xtail)


def _sc_gather(idx2, emb_lin, lin_flat):
    mesh = plsc.VectorSubcoreMesh(core_axis_name="c", subcore_axis_name="s")

    @functools.partial(
        pl.kernel,
        out_type=(
            jax.ShapeDtypeStruct((NBLK, ROWS_PER_STREAM, EMBED_DIM), jnp.float32),
            jax.ShapeDtypeStruct((NBLK, ROWS_PER_STREAM), jnp.float32),
        ),
        mesh=mesh,
        scratch_types=[
            pltpu.VMEM((CHUNK_BLKS, ROWS_PER_STREAM), jnp.int32),
            pltpu.VMEM((CHUNK_BLKS, ROWS_PER_STREAM, EMBED_DIM), jnp.float32),
            pltpu.VMEM((CHUNK_BLKS, ROWS_PER_STREAM), jnp.float32),
            pltpu.SemaphoreType.DMA,
            pltpu.SemaphoreType.DMA,
        ],
        compiler_params=pltpu.CompilerParams(use_tc_tiling_on_sc=False),
    )
    def k(idx_hbm, emb_hbm, lin_hbm, emb_out, lin_out, idx_v, rows_v, lin_v,
          sem_e, sem_l):
        wid = lax.axis_index("s") * NC + lax.axis_index("c")
        base = wid * BLK_PER_W

        def body(c, carry):
            off = base + c * CHUNK_BLKS
            pltpu.sync_copy(idx_hbm.at[pl.ds(off, CHUNK_BLKS)], idx_v)
            embs = [pltpu.async_copy(emb_hbm.at[idx_v.at[j]], rows_v.at[j], sem_e)
                    for j in range(CHUNK_BLKS)]
            lins = [pltpu.async_copy(lin_hbm.at[idx_v.at[j]], lin_v.at[j], sem_l)
                    for j in range(CHUNK_BLKS)]
            for cp in embs:
                cp.wait()
            for cp in lins:
                cp.wait()
            pltpu.sync_copy(rows_v, emb_out.at[pl.ds(off, CHUNK_BLKS)])
            pltpu.sync_copy(lin_v, lin_out.at[pl.ds(off, CHUNK_BLKS)])
            return carry

        lax.fori_loop(0, NCHUNK, body, 0)

    return k(idx2, emb_lin, lin_flat)


BB = 1024
MB = BB // 8
GRID = BATCH // BB


def _tc_body(embp_ref, linp_ref, bd1_ref, b1t_ref, bd2_ref, b2t_ref, bd3_ref,
             ones_ref, cb_ref, out_ref):
    e = embp_ref[...]
    s = jnp.sum(e, axis=0)
    s2 = jnp.sum(e * e, axis=0)
    g = s * s - s2
    fm = 0.5 * jnp.dot(g.astype(jnp.bfloat16), ones_ref[...],
                       preferred_element_type=jnp.float32)
    eb = e.reshape(NUM_FIELDS * MB, 128).astype(jnp.bfloat16)
    h = jnp.dot(eb, bd1_ref[...], preferred_element_type=jnp.float32)
    h = jnp.maximum(h + b1t_ref[...], 0.0)
    h2 = jnp.dot(h.astype(jnp.bfloat16), bd2_ref[...],
                 preferred_element_type=jnp.float32)
    h2 = jnp.maximum(h2 + b2t_ref[...], 0.0)
    h2s = jnp.sum(h2.reshape(NUM_FIELDS, MB, 256), axis=0)
    mlp = jnp.dot(h2s.astype(jnp.bfloat16), bd3_ref[...],
                  preferred_element_type=jnp.float32)
    lin = jnp.sum(linp_ref[...], axis=0)
    logits = lin + fm + mlp + cb_ref[0, 0]
    out_ref[...] = jax.nn.sigmoid(logits)


def _tc_dense(embp, linp, bd1, b1t, bd2, b2t, bd3, onesbd, cb):
    return pl.pallas_call(
        _tc_body,
        grid=(GRID,),
        in_specs=[
            pl.BlockSpec((NUM_FIELDS, MB, 128), lambda i: (0, i, 0)),
            pl.BlockSpec((NUM_FIELDS, MB, 8), lambda i: (0, i, 0)),
            pl.BlockSpec((128, 512), lambda i: (0, 0)),
            pl.BlockSpec((1, 512), lambda i: (0, 0)),
            pl.BlockSpec((512, 256), lambda i: (0, 0)),
            pl.BlockSpec((1, 256), lambda i: (0, 0)),
            pl.BlockSpec((256, 8), lambda i: (0, 0)),
            pl.BlockSpec((128, 8), lambda i: (0, 0)),
            pl.BlockSpec((1, 1), lambda i: (0, 0)),
        ],
        out_specs=pl.BlockSpec((MB, 8), lambda i: (i, 0)),
        out_shape=jax.ShapeDtypeStruct((BATCH // 8, 8), jnp.float32),
    )(embp, linp, bd1, b1t, bd2, b2t, bd3, onesbd, cb)


def kernel(x, emb_table, lin_table, lin_bias, W1, b1, W2, b2, W3, b3):
    offsets = FIELD_DIM * jnp.arange(NUM_FIELDS, dtype=jnp.int32)
    idx = (x.astype(jnp.int32).T + offsets[:, None]).reshape(NBLK, ROWS_PER_STREAM)
    lin_flat = lin_table.reshape(-1)

    xtail = emb_table[XTAIL_START:VOCAB].reshape(XTAIL * EMBED_DIM)
    emb_lin = _sc_transpose(emb_table.T, xtail).reshape(VPAD, EMBED_DIM)
    emb_rows, lin_rows = _sc_gather(idx, emb_lin, lin_flat)
    embp = emb_rows.reshape(NUM_FIELDS, BATCH // 8, 128)
    linp = lin_rows.reshape(NUM_FIELDS, BATCH // 8, 8)

    eye8 = jnp.eye(8, dtype=jnp.float32)
    bd1 = jnp.kron(eye8, W1).astype(jnp.bfloat16)
    bd2 = jnp.kron(eye8, W2).astype(jnp.bfloat16)
    bd3 = jnp.kron(eye8, W3).astype(jnp.bfloat16)
    onesbd = jnp.kron(eye8, jnp.ones((EMBED_DIM, 1), jnp.float32)).astype(
        jnp.bfloat16)
    b1t = jnp.tile(b1, 8).reshape(1, 512)
    b2t = jnp.tile(b2, 8).reshape(1, 256)
    cb = (lin_bias[0] + NUM_FIELDS * b3[0]).reshape(1, 1)

    out8 = _tc_dense(embp, linp, bd1, b1t, bd2, b2t, bd3, onesbd, cb)
    return out8.reshape(BATCH)
